# Initial kernel scaffold; baseline (speedup 1.0000x reference)
import functools

import jax
import jax.numpy as jnp
from jax import lax
from jax.experimental import pallas as pl
from jax.experimental.pallas import tpu as pltpu
from jax.experimental.pallas import tpu_sc as plsc

N = 10000
E = 320000
D = 128
EPS = 1e-5

NC = 2
NS = 16
NW = NC * NS
ET = E // NW
CW = 100
CH = ET // CW
RT = N // NS
ZR = 125

_sc_mesh = plsc.VectorSubcoreMesh(core_axis_name="c", subcore_axis_name="s")
_sc_params = pltpu.CompilerParams(needs_layout_passes=False)


@functools.partial(
    pl.kernel,
    out_type=jax.ShapeDtypeStruct((NW, N), jnp.float32),
    mesh=_sc_mesh,
    compiler_params=_sc_params,
    scratch_types=[
        pltpu.VMEM((ET,), jnp.int32),
        pltpu.VMEM((N,), jnp.float32),
    ],
)
def _sc_count(dst_hbm, cnt_out, dstv, deg):
    c = lax.axis_index("c")
    s = lax.axis_index("s")
    w = c * NS + s
    pltpu.sync_copy(dst_hbm.at[w], dstv)
    zeros = jnp.zeros((16,), jnp.float32)

    def zbody(i, carry):
        deg[pl.ds(i * 16, 16)] = zeros
        return carry

    lax.fori_loop(0, N // 16, zbody, 0)
    ones = jnp.full((16,), 1.0, jnp.float32)

    def cbody(j, carry):
        d16 = dstv[pl.ds(j * 16, 16)]
        plsc.addupdate_scatter(deg, [d16], ones)
        return carry

    lax.fori_loop(0, ET // 16, cbody, 0)
    pltpu.sync_copy(deg, cnt_out.at[w])


@functools.partial(
    pl.kernel,
    out_type=jax.ShapeDtypeStruct((NC, N, D), jnp.float32),
    mesh=_sc_mesh,
    compiler_params=_sc_params,
    scratch_types=[
        pltpu.VMEM((CH, CW), jnp.int32),
        pltpu.VMEM((CH, CW), jnp.int32),
        pltpu.VMEM((CW, D), jnp.float32),
        pltpu.VMEM((ZR, D), jnp.float32),
        pltpu.VMEM_SHARED((N, D), jnp.float32),
        pltpu.SemaphoreType.DMA,
    ],
)
def _sc_scatter(src_hbm, dst_hbm, g_hbm, acc_out, srcv, dstv, rows, zbuf,
                acc_sh, sem):
    c = lax.axis_index("c")
    s = lax.axis_index("s")
    w = c * NS + s
    pltpu.sync_copy(src_hbm.at[w], srcv)
    pltpu.sync_copy(dst_hbm.at[w], dstv)
    zeros = jnp.zeros((16,), jnp.float32)

    def zbody(i, carry):
        for k in range(D // 16):
            zbuf[i, pl.ds(k * 16, 16)] = zeros
        return carry

    lax.fori_loop(0, ZR, zbody, 0)
    for r in range(RT // ZR):
        pltpu.sync_copy(zbuf, acc_sh.at[pl.ds(s * RT + r * ZR, ZR)])
    plsc.subcore_barrier()

    def body(j, carry):
        pltpu.async_copy(g_hbm.at[srcv.at[j]], rows, sem).wait()
        pltpu.sync_copy(rows, acc_sh.at[dstv.at[j]], add=True)
        return carry

    lax.fori_loop(0, CH, body, 0)
    plsc.subcore_barrier()
    pltpu.sync_copy(acc_sh.at[pl.ds(s * RT, RT)],
                    acc_out.at[c, pl.ds(s * RT, RT)])


_BN = 1000


def _tc_prep_body(x_ref, w_ref, cnt_ref, g_ref, dinv_ref):
    deg = jnp.sum(cnt_ref[...], axis=0, keepdims=True) + 1.0
    dinv = lax.rsqrt(deg)
    h = jnp.dot(x_ref[...], w_ref[...], preferred_element_type=jnp.float32)
    g_ref[...] = h * dinv.T
    dinv_ref[...] = dinv.T


def _tc_prep(x, W, counts):
    return pl.pallas_call(
        _tc_prep_body,
        grid=(N // _BN,),
        in_specs=[
            pl.BlockSpec((_BN, D), lambda i: (i, 0)),
            pl.BlockSpec((D, D), lambda i: (0, 0)),
            pl.BlockSpec((NW, _BN), lambda i: (0, i)),
        ],
        out_specs=[
            pl.BlockSpec((_BN, D), lambda i: (i, 0)),
            pl.BlockSpec((_BN, 1), lambda i: (i, 0)),
        ],
        out_shape=[
            jax.ShapeDtypeStruct((N, D), jnp.float32),
            jax.ShapeDtypeStruct((N, 1), jnp.float32),
        ],
    )(x, W, counts)


def _tc_epi_body(g_ref, a0_ref, a1_ref, dinv_ref, b_ref, gam_ref, bet_ref,
                 x_ref, o_ref):
    tot = (g_ref[...] + a0_ref[...] + a1_ref[...]) * dinv_ref[...]
    tot = tot + b_ref[...]
    mu = jnp.mean(tot, axis=1, keepdims=True)
    cen = tot - mu
    var = jnp.mean(cen * cen, axis=1, keepdims=True)
    xh = cen * lax.rsqrt(var + EPS)
    y = xh * gam_ref[...] + bet_ref[...]
    o_ref[...] = jnp.maximum(y + x_ref[...], 0.0)


def _tc_epilogue(g, a0, a1, dinv, b, gamma, beta, x):
    vec = pl.BlockSpec((1, D), lambda i: (0, 0))
    blk = pl.BlockSpec((_BN, D), lambda i: (i, 0))
    return pl.pallas_call(
        _tc_epi_body,
        grid=(N // _BN,),
        in_specs=[blk, blk, blk,
                  pl.BlockSpec((_BN, 1), lambda i: (i, 0)),
                  vec, vec, vec, blk],
        out_specs=blk,
        out_shape=jax.ShapeDtypeStruct((N, D), jnp.float32),
    )(g, a0, a1, dinv, b.reshape(1, D), gamma.reshape(1, D),
      beta.reshape(1, D), x)


def kernel(x, edge_index, W, b, gamma, beta):
    src = edge_index[0].reshape(NW, CH, CW)
    dst = edge_index[1].reshape(NW, CH, CW)
    dst_by_tile = edge_index[1].reshape(NW, ET)
    counts = _sc_count(dst_by_tile)
    g, dinv = _tc_prep(x, W, counts)
    acc = _sc_scatter(src, dst, g)
    return _tc_epilogue(g, acc[0], acc[1], dinv, b, gamma, beta, x)


# speedup vs baseline: 27.9628x; 27.9628x over previous
import functools

import jax
import jax.numpy as jnp
from jax import lax
from jax.experimental import pallas as pl
from jax.experimental.pallas import tpu as pltpu
from jax.experimental.pallas import tpu_sc as plsc

N = 10000
E = 320000
D = 128
EPS = 1e-5

NC = 2
NS = 16
NW = NC * NS
CW = 128
CH = 80
ET = CH * CW
EP = NW * ET
NP = 10240
RT = NP // NS
ZR = 128

_sc_mesh = plsc.VectorSubcoreMesh(core_axis_name="c", subcore_axis_name="s")
_sc_params = pltpu.CompilerParams(needs_layout_passes=False)


@functools.partial(
    pl.kernel,
    out_type=jax.ShapeDtypeStruct((NW * NP,), jnp.float32),
    mesh=_sc_mesh,
    compiler_params=_sc_params,
    scratch_types=[
        pltpu.VMEM((CH, CW), jnp.int32),
        pltpu.VMEM((NP,), jnp.float32),
    ],
)
def _sc_count(dst_hbm, cnt_out, dstv, deg):
    c = lax.axis_index("c")
    s = lax.axis_index("s")
    w = c * NS + s
    pltpu.sync_copy(dst_hbm.at[pl.ds(w * CH, CH)], dstv)
    zeros = jnp.zeros((16,), jnp.float32)

    def zbody(i, carry):
        deg[pl.ds(i * 16, 16)] = zeros
        return carry

    lax.fori_loop(0, NP // 16, zbody, 0)
    ones = jnp.full((16,), 1.0, jnp.float32)

    def cbody(j, carry):
        d16 = dstv[j >> 3, pl.ds((j & 7) * 16, 16)]
        plsc.addupdate_scatter(deg, [d16], ones)
        return carry

    lax.fori_loop(0, ET // 16, cbody, 0)
    pltpu.sync_copy(deg, cnt_out.at[pl.ds(w * NP, NP)])


@functools.partial(
    pl.kernel,
    out_type=jax.ShapeDtypeStruct((NC, NP, D), jnp.float32),
    mesh=_sc_mesh,
    compiler_params=_sc_params,
    scratch_types=[
        pltpu.VMEM((CH, CW), jnp.int32),
        pltpu.VMEM((CH, CW), jnp.int32),
        pltpu.VMEM((CW, D), jnp.float32),
        pltpu.VMEM_SHARED((NP, D), jnp.float32),
        pltpu.SemaphoreType.DMA,
    ],
)
def _sc_scatter(src_hbm, dst_hbm, g_hbm, acc_out, srcv, dstv, rows,
                acc_sh, sem):
    c = lax.axis_index("c")
    s = lax.axis_index("s")
    w = c * NS + s
    pltpu.sync_copy(src_hbm.at[pl.ds(w * CH, CH)], srcv)
    pltpu.sync_copy(dst_hbm.at[pl.ds(w * CH, CH)], dstv)
    zeros = jnp.zeros((16,), jnp.float32)

    def zbody(i, carry):
        for k in range(D // 16):
            rows[i, pl.ds(k * 16, 16)] = zeros
        return carry

    lax.fori_loop(0, ZR, zbody, 0)
    for r in range(RT // ZR):
        pltpu.sync_copy(rows, acc_sh.at[pl.ds(s * RT + r * ZR, ZR)])
    plsc.subcore_barrier()

    def body(j, carry):
        pltpu.async_copy(g_hbm.at[srcv.at[j]], rows, sem).wait()
        pltpu.sync_copy(rows, acc_sh.at[dstv.at[j]], add=True)
        return carry

    lax.fori_loop(0, CH, body, 0)
    plsc.subcore_barrier()
    pltpu.sync_copy(acc_sh.at[pl.ds(s * RT, RT)],
                    acc_out.at[c, pl.ds(s * RT, RT)])


def _tc_prep_body(x_ref, w_ref, cnt_ref, g_ref, dinv_ref):
    deg = jnp.sum(cnt_ref[...], axis=0, keepdims=True) + 1.0
    dinv = jnp.transpose(lax.rsqrt(deg[:, :N]))
    h = jnp.dot(x_ref[...], w_ref[...], preferred_element_type=jnp.float32)
    g_ref[...] = h * dinv
    dinv_ref[...] = dinv


def _tc_prep(x, W, counts):
    return pl.pallas_call(
        _tc_prep_body,
        out_shape=[
            jax.ShapeDtypeStruct((N, D), jnp.float32),
            jax.ShapeDtypeStruct((N, 1), jnp.float32),
        ],
    )(x, W, counts)


def _tc_epi_body(g_ref, a0_ref, a1_ref, dinv_ref, b_ref, gam_ref, bet_ref,
                 x_ref, o_ref):
    tot = (g_ref[...] + a0_ref[...] + a1_ref[...]) * dinv_ref[...]
    tot = tot + b_ref[...]
    mu = jnp.mean(tot, axis=1, keepdims=True)
    cen = tot - mu
    var = jnp.mean(cen * cen, axis=1, keepdims=True)
    xh = cen * lax.rsqrt(var + EPS)
    y = xh * gam_ref[...] + bet_ref[...]
    o_ref[...] = jnp.maximum(y + x_ref[...], 0.0)


def _tc_epilogue(g, a0, a1, dinv, b, gamma, beta, x):
    return pl.pallas_call(
        _tc_epi_body,
        out_shape=jax.ShapeDtypeStruct((N, D), jnp.float32),
    )(g, a0, a1, dinv, b.reshape(1, D), gamma.reshape(1, D),
      beta.reshape(1, D), x)


def kernel(x, edge_index, W, b, gamma, beta):
    pad = EP - E
    ar = jnp.arange(pad, dtype=jnp.int32)
    src_pad = jnp.concatenate([edge_index[0], (ar * 131) % N])
    dst_pad = jnp.concatenate([edge_index[1], N + ar % (NP - N)])
    src2 = src_pad.reshape(NW * CH, CW)
    dst2 = dst_pad.reshape(NW * CH, CW)
    counts = _sc_count(dst2).reshape(NW, NP)
    g, dinv = _tc_prep(x, W, counts)
    acc = _sc_scatter(src2, dst2, g)
    return _tc_epilogue(g, acc[0, :N], acc[1, :N], dinv, b, gamma, beta, x)


# speedup vs baseline: 38.3135x; 1.3702x over previous
import functools

import jax
import jax.numpy as jnp
from jax import lax
from jax.experimental import pallas as pl
from jax.experimental.pallas import tpu as pltpu
from jax.experimental.pallas import tpu_sc as plsc

N = 10000
E = 320000
D = 128
EPS = 1e-5

NC = 2
NS = 16
NW = NC * NS
CW = 128
CH = 80
ET = CH * CW
EP = NW * ET
NP = 10240
RT = NP // NS
ZR = 128

_sc_mesh = plsc.VectorSubcoreMesh(core_axis_name="c", subcore_axis_name="s")
_sc_params = pltpu.CompilerParams(needs_layout_passes=False)


@functools.partial(
    pl.kernel,
    out_type=jax.ShapeDtypeStruct((NW * NP,), jnp.float32),
    mesh=_sc_mesh,
    compiler_params=_sc_params,
    scratch_types=[
        pltpu.VMEM((CH, CW), jnp.int32),
        pltpu.VMEM((NP,), jnp.float32),
    ],
)
def _sc_count(dst_hbm, cnt_out, dstv, deg):
    c = lax.axis_index("c")
    s = lax.axis_index("s")
    w = c * NS + s
    pltpu.sync_copy(dst_hbm.at[pl.ds(w * CH, CH)], dstv)
    zeros = jnp.zeros((16,), jnp.float32)

    def zbody(i, carry):
        deg[pl.ds(i * 16, 16)] = zeros
        return carry

    lax.fori_loop(0, NP // 16, zbody, 0)
    ones = jnp.full((16,), 1.0, jnp.float32)

    def cbody(j, carry):
        d16 = dstv[j >> 3, pl.ds((j & 7) * 16, 16)]
        plsc.addupdate_scatter(deg, [d16], ones)
        return carry

    lax.fori_loop(0, ET // 16, cbody, 0)
    pltpu.sync_copy(deg, cnt_out.at[pl.ds(w * NP, NP)])


@functools.partial(
    pl.kernel,
    out_type=jax.ShapeDtypeStruct((NC, NP, D), jnp.float32),
    mesh=_sc_mesh,
    compiler_params=_sc_params,
    scratch_types=[
        pltpu.VMEM((CH // 2, CW), jnp.int32),
        pltpu.VMEM((CH // 2, CW), jnp.int32),
        pltpu.VMEM((CW, D), jnp.float32),
        pltpu.VMEM((CW, D), jnp.float32),
        pltpu.VMEM_SHARED((NP, D), jnp.float32),
        pltpu.SemaphoreType.DMA,
        pltpu.SemaphoreType.DMA,
    ],
)
def _sc_scatter(src_hbm, dst_hbm, g_hbm, acc_out, srcv, dstv, r0, r1,
                acc_sh, sem0, sem1):
    c = lax.axis_index("c")
    s = lax.axis_index("s")
    w = c * NS + s
    zeros = jnp.zeros((16,), jnp.float32)

    def zbody(i, carry):
        for k in range(D // 16):
            r0[i, pl.ds(k * 16, 16)] = zeros
        return carry

    lax.fori_loop(0, ZR, zbody, 0)
    for r in range(RT // ZR):
        pltpu.sync_copy(r0, acc_sh.at[pl.ds(s * RT + r * ZR, ZR)])
    plsc.subcore_barrier()

    CHH = CH // 2
    for p in range(2):
        pltpu.sync_copy(src_hbm.at[pl.ds(w * CH + p * CHH, CHH)], srcv)
        pltpu.sync_copy(dst_hbm.at[pl.ds(w * CH + p * CHH, CHH)], dstv)
        pltpu.async_copy(g_hbm.at[srcv.at[0]], r0, sem0)

        def body(i, carry):
            j0 = 2 * i
            j1 = 2 * i + 1
            pltpu.async_copy(g_hbm.at[srcv.at[j1]], r1, sem1)
            pltpu.make_async_copy(g_hbm.at[srcv.at[j0]], r0, sem0).wait()
            pltpu.sync_copy(r0, acc_sh.at[dstv.at[j0]], add=True)

            @pl.when(j1 + 1 < CHH)
            def _():
                pltpu.async_copy(g_hbm.at[srcv.at[j1 + 1]], r0, sem0)

            pltpu.make_async_copy(g_hbm.at[srcv.at[j1]], r1, sem1).wait()
            pltpu.sync_copy(r1, acc_sh.at[dstv.at[j1]], add=True)
            return carry

        lax.fori_loop(0, CHH // 2, body, 0)
    plsc.subcore_barrier()
    pltpu.sync_copy(acc_sh.at[pl.ds(s * RT, RT)],
                    acc_out.at[c, pl.ds(s * RT, RT)])


def _tc_prep_body(x_ref, w_ref, cnt_ref, g_ref, dinv_ref):
    deg = jnp.sum(cnt_ref[...], axis=0, keepdims=True) + 1.0
    dinv = jnp.transpose(lax.rsqrt(deg[:, :N]))
    h = jnp.dot(x_ref[...], w_ref[...], preferred_element_type=jnp.float32)
    g_ref[...] = h * dinv
    dinv_ref[...] = dinv


def _tc_prep(x, W, counts):
    return pl.pallas_call(
        _tc_prep_body,
        out_shape=[
            jax.ShapeDtypeStruct((N, D), jnp.float32),
            jax.ShapeDtypeStruct((N, 1), jnp.float32),
        ],
    )(x, W, counts)


def _tc_epi_body(g_ref, a0_ref, a1_ref, dinv_ref, b_ref, gam_ref, bet_ref,
                 x_ref, o_ref):
    tot = (g_ref[...] + a0_ref[...] + a1_ref[...]) * dinv_ref[...]
    tot = tot + b_ref[...]
    mu = jnp.mean(tot, axis=1, keepdims=True)
    cen = tot - mu
    var = jnp.mean(cen * cen, axis=1, keepdims=True)
    xh = cen * lax.rsqrt(var + EPS)
    y = xh * gam_ref[...] + bet_ref[...]
    o_ref[...] = jnp.maximum(y + x_ref[...], 0.0)


def _tc_epilogue(g, a0, a1, dinv, b, gamma, beta, x):
    return pl.pallas_call(
        _tc_epi_body,
        out_shape=jax.ShapeDtypeStruct((N, D), jnp.float32),
    )(g, a0, a1, dinv, b.reshape(1, D), gamma.reshape(1, D),
      beta.reshape(1, D), x)


def kernel(x, edge_index, W, b, gamma, beta):
    pad = EP - E
    ar = jnp.arange(pad, dtype=jnp.int32)
    src_pad = jnp.concatenate([edge_index[0], (ar * 131) % N])
    dst_pad = jnp.concatenate([edge_index[1], N + ar % (NP - N)])
    src2 = src_pad.reshape(NW * CH, CW)
    dst2 = dst_pad.reshape(NW * CH, CW)
    counts = _sc_count(dst2).reshape(NW, NP)
    g, dinv = _tc_prep(x, W, counts)
    acc = _sc_scatter(src2, dst2, g)
    return _tc_epilogue(g, acc[0, :N], acc[1, :N], dinv, b, gamma, beta, x)


# speedup vs baseline: 39.5377x; 1.0320x over previous
import functools

import jax
import jax.numpy as jnp
from jax import lax
from jax.experimental import pallas as pl
from jax.experimental.pallas import tpu as pltpu
from jax.experimental.pallas import tpu_sc as plsc

N = 10000
E = 320000
D = 128
EPS = 1e-5

NC = 2
NS = 16
NW = NC * NS
CW = 128
CH = 80
ET = CH * CW
EP = NW * ET
NP = 10240
RT = NP // NS
ZR = 128

_sc_mesh = plsc.VectorSubcoreMesh(core_axis_name="c", subcore_axis_name="s")
_sc_params = pltpu.CompilerParams(needs_layout_passes=False)


@functools.partial(
    pl.kernel,
    out_type=jax.ShapeDtypeStruct((NW * NP,), jnp.float32),
    mesh=_sc_mesh,
    compiler_params=_sc_params,
    scratch_types=[
        pltpu.VMEM((CH, CW), jnp.int32),
        pltpu.VMEM((NP,), jnp.float32),
    ],
)
def _sc_count(dst_hbm, cnt_out, dstv, deg):
    c = lax.axis_index("c")
    s = lax.axis_index("s")
    w = c * NS + s
    pltpu.sync_copy(dst_hbm.at[pl.ds(w * CH, CH)], dstv)
    zeros = jnp.zeros((16,), jnp.float32)

    def zbody(i, carry):
        deg[pl.ds(i * 16, 16)] = zeros
        return carry

    lax.fori_loop(0, NP // 16, zbody, 0)
    ones = jnp.full((16,), 1.0, jnp.float32)

    def cbody(j, carry):
        d16 = dstv[j >> 3, pl.ds((j & 7) * 16, 16)]
        plsc.addupdate_scatter(deg, [d16], ones)
        return carry

    lax.fori_loop(0, ET // 16, cbody, 0)
    pltpu.sync_copy(deg, cnt_out.at[pl.ds(w * NP, NP)])


@functools.partial(
    pl.kernel,
    out_type=jax.ShapeDtypeStruct((NC, NP, D), jnp.float32),
    mesh=_sc_mesh,
    compiler_params=_sc_params,
    scratch_types=[
        pltpu.VMEM((CH // 2, CW), jnp.int32),
        pltpu.VMEM((CH // 2, CW), jnp.int32),
        pltpu.VMEM((CW, D), jnp.float32),
        pltpu.VMEM((CW, D), jnp.float32),
        pltpu.VMEM_SHARED((NP, D), jnp.float32),
        pltpu.SemaphoreType.DMA,
        pltpu.SemaphoreType.DMA,
    ],
)
def _sc_scatter(src_hbm, dst_hbm, g_hbm, acc_out, srcv, dstv, r0, r1,
                acc_sh, sem0, sem1):
    c = lax.axis_index("c")
    s = lax.axis_index("s")
    w = c * NS + s
    zeros = jnp.zeros((16,), jnp.float32)

    def zbody(i, carry):
        for k in range(D // 16):
            r0[i, pl.ds(k * 16, 16)] = zeros
        return carry

    lax.fori_loop(0, ZR, zbody, 0)
    for r in range(RT // ZR):
        pltpu.sync_copy(r0, acc_sh.at[pl.ds(s * RT + r * ZR, ZR)])
    plsc.subcore_barrier()

    CHH = CH // 2
    for p in range(2):
        pltpu.sync_copy(src_hbm.at[pl.ds(w * CH + p * CHH, CHH)], srcv)
        pltpu.sync_copy(dst_hbm.at[pl.ds(w * CH + p * CHH, CHH)], dstv)
        pltpu.async_copy(g_hbm.at[srcv.at[0]], r0, sem0)

        def body(i, carry):
            j0 = 2 * i
            j1 = 2 * i + 1
            pltpu.async_copy(g_hbm.at[srcv.at[j1]], r1, sem1)
            pltpu.make_async_copy(g_hbm.at[srcv.at[j0]], r0, sem0).wait()
            pltpu.sync_copy(r0, acc_sh.at[dstv.at[j0]], add=True)

            @pl.when(j1 + 1 < CHH)
            def _():
                pltpu.async_copy(g_hbm.at[srcv.at[j1 + 1]], r0, sem0)

            pltpu.make_async_copy(g_hbm.at[srcv.at[j1]], r1, sem1).wait()
            pltpu.sync_copy(r1, acc_sh.at[dstv.at[j1]], add=True)
            return carry

        lax.fori_loop(0, CHH // 2, body, 0)
    plsc.subcore_barrier()
    pltpu.sync_copy(acc_sh.at[pl.ds(s * RT, RT)],
                    acc_out.at[c, pl.ds(s * RT, RT)])


def _tc_mm_body(x_ref, w_ref, h_ref):
    h_ref[...] = jnp.dot(x_ref[...], w_ref[...],
                         preferred_element_type=jnp.float32)


def _tc_mm(x, W):
    return pl.pallas_call(
        _tc_mm_body,
        out_shape=jax.ShapeDtypeStruct((N, D), jnp.float32),
    )(x, W)


def _tc_scale_body(h_ref, cnt_ref, g_ref, dinv_ref):
    deg = jnp.sum(cnt_ref[...], axis=0, keepdims=True) + 1.0
    dinv = jnp.transpose(lax.rsqrt(deg[:, :N]))
    g_ref[...] = h_ref[...] * dinv
    dinv_ref[...] = dinv


def _tc_scale(h, counts):
    return pl.pallas_call(
        _tc_scale_body,
        out_shape=[
            jax.ShapeDtypeStruct((N, D), jnp.float32),
            jax.ShapeDtypeStruct((N, 1), jnp.float32),
        ],
    )(h, counts)


def _tc_epi_body(g_ref, acc_ref, dinv_ref, b_ref, gam_ref, bet_ref,
                 x_ref, o_ref):
    tot = g_ref[...] + acc_ref[0, :N, :] + acc_ref[1, :N, :]
    tot = tot * dinv_ref[...] + b_ref[...]
    mu = jnp.mean(tot, axis=1, keepdims=True)
    cen = tot - mu
    var = jnp.mean(cen * cen, axis=1, keepdims=True)
    xh = cen * lax.rsqrt(var + EPS)
    y = xh * gam_ref[...] + bet_ref[...]
    o_ref[...] = jnp.maximum(y + x_ref[...], 0.0)


def _tc_epilogue(g, acc, dinv, b, gamma, beta, x):
    return pl.pallas_call(
        _tc_epi_body,
        out_shape=jax.ShapeDtypeStruct((N, D), jnp.float32),
    )(g, acc, dinv, b.reshape(1, D), gamma.reshape(1, D),
      beta.reshape(1, D), x)


def kernel(x, edge_index, W, b, gamma, beta):
    pad = EP - E
    ar = jnp.arange(pad, dtype=jnp.int32)
    src_pad = jnp.concatenate([edge_index[0], (ar * 131) % N])
    dst_pad = jnp.concatenate([edge_index[1], N + ar % (NP - N)])
    src2 = src_pad.reshape(NW * CH, CW)
    dst2 = dst_pad.reshape(NW * CH, CW)
    counts = _sc_count(dst2).reshape(NW, NP)
    h = _tc_mm(x, W)
    g, dinv = _tc_scale(h, counts)
    acc = _sc_scatter(src2, dst2, g)
    return _tc_epilogue(g, acc, dinv, b, gamma, beta, x)


# speedup vs baseline: 39.9174x; 1.0096x over previous
import functools

import jax
import jax.numpy as jnp
from jax import lax
from jax.experimental import pallas as pl
from jax.experimental.pallas import tpu as pltpu
from jax.experimental.pallas import tpu_sc as plsc

N = 10000
E = 320000
D = 128
EPS = 1e-5

NC = 2
NS = 16
NW = NC * NS
CW = 128
CH = 80
ET = CH * CW
EP = NW * ET
NP = 10240
RT = NP // NS
ZR = 128

_sc_mesh = plsc.VectorSubcoreMesh(core_axis_name="c", subcore_axis_name="s")
_sc_params = pltpu.CompilerParams(needs_layout_passes=False)


@functools.partial(
    pl.kernel,
    out_type=jax.ShapeDtypeStruct((NW * NP,), jnp.float32),
    mesh=_sc_mesh,
    compiler_params=_sc_params,
    scratch_types=[
        pltpu.VMEM((CH, CW), jnp.int32),
        pltpu.VMEM((NP,), jnp.float32),
    ],
)
def _sc_count(dst_hbm, cnt_out, dstv, deg):
    c = lax.axis_index("c")
    s = lax.axis_index("s")
    w = c * NS + s
    pltpu.sync_copy(dst_hbm.at[pl.ds(w * CH, CH)], dstv)
    zeros = jnp.zeros((16,), jnp.float32)

    def zbody(i, carry):
        deg[pl.ds(i * 16, 16)] = zeros
        return carry

    lax.fori_loop(0, NP // 16, zbody, 0)
    ones = jnp.full((16,), 1.0, jnp.float32)

    def cbody(j, carry):
        d16 = dstv[j >> 3, pl.ds((j & 7) * 16, 16)]
        plsc.addupdate_scatter(deg, [d16], ones)
        return carry

    lax.fori_loop(0, ET // 16, cbody, 0)
    pltpu.sync_copy(deg, cnt_out.at[pl.ds(w * NP, NP)])


@functools.partial(
    pl.kernel,
    out_type=jax.ShapeDtypeStruct((NC, NP, D), jnp.float32),
    mesh=_sc_mesh,
    compiler_params=_sc_params,
    scratch_types=[
        pltpu.VMEM((CH // 2, CW), jnp.int32),
        pltpu.VMEM((CH // 2, CW), jnp.int32),
        pltpu.VMEM((CW, D), jnp.float32),
        pltpu.VMEM((CW, D), jnp.float32),
        pltpu.VMEM_SHARED((NP, D), jnp.float32),
        pltpu.SemaphoreType.DMA,
        pltpu.SemaphoreType.DMA,
    ],
)
def _sc_scatter(src_hbm, dst_hbm, g_hbm, acc_out, srcv, dstv, r0, r1,
                acc_sh, sem0, sem1):
    c = lax.axis_index("c")
    s = lax.axis_index("s")
    w = c * NS + s
    zeros = jnp.zeros((16,), jnp.float32)

    CHH = CH // 2
    pltpu.sync_copy(src_hbm.at[pl.ds(w * CH, CHH)], srcv)
    pltpu.sync_copy(dst_hbm.at[pl.ds(w * CH, CHH)], dstv)
    pltpu.async_copy(g_hbm.at[srcv.at[0]], r1, sem1)

    def zbody(i, carry):
        for k in range(D // 16):
            r0[i, pl.ds(k * 16, 16)] = zeros
        return carry

    lax.fori_loop(0, ZR, zbody, 0)
    for r in range(RT // ZR):
        pltpu.sync_copy(r0, acc_sh.at[pl.ds(s * RT + r * ZR, ZR)])
    plsc.subcore_barrier()

    def body(i, carry):
        j0 = 2 * i
        j1 = 2 * i + 1
        pltpu.async_copy(g_hbm.at[srcv.at[j1]], r0, sem0)
        pltpu.make_async_copy(g_hbm.at[srcv.at[j0]], r1, sem1).wait()
        pltpu.sync_copy(r1, acc_sh.at[dstv.at[j0]], add=True)

        @pl.when(j1 + 1 < CHH)
        def _():
            pltpu.async_copy(g_hbm.at[srcv.at[j1 + 1]], r1, sem1)

        pltpu.make_async_copy(g_hbm.at[srcv.at[j1]], r0, sem0).wait()
        pltpu.sync_copy(r0, acc_sh.at[dstv.at[j1]], add=True)
        return carry

    lax.fori_loop(0, CHH // 2, body, 0)
    pltpu.sync_copy(src_hbm.at[pl.ds(w * CH + CHH, CHH)], srcv)
    pltpu.sync_copy(dst_hbm.at[pl.ds(w * CH + CHH, CHH)], dstv)
    pltpu.async_copy(g_hbm.at[srcv.at[0]], r1, sem1)
    lax.fori_loop(0, CHH // 2, body, 0)
    plsc.subcore_barrier()
    pltpu.sync_copy(acc_sh.at[pl.ds(s * RT, RT)],
                    acc_out.at[c, pl.ds(s * RT, RT)])


def _tc_mm_body(x_ref, w_ref, h_ref):
    h_ref[...] = jnp.dot(x_ref[...], w_ref[...],
                         preferred_element_type=jnp.float32)


def _tc_mm(x, W):
    return pl.pallas_call(
        _tc_mm_body,
        out_shape=jax.ShapeDtypeStruct((N, D), jnp.float32),
    )(x, W)


def _tc_scale_body(h_ref, cnt_ref, g_ref, dinv_ref):
    deg = jnp.sum(cnt_ref[...], axis=0, keepdims=True) + 1.0
    dinv = jnp.transpose(lax.rsqrt(deg[:, :N]))
    g_ref[...] = h_ref[...] * dinv
    dinv_ref[...] = dinv


def _tc_scale(h, counts):
    return pl.pallas_call(
        _tc_scale_body,
        out_shape=[
            jax.ShapeDtypeStruct((N, D), jnp.float32),
            jax.ShapeDtypeStruct((N, 1), jnp.float32),
        ],
    )(h, counts)


def _tc_epi_body(g_ref, acc_ref, dinv_ref, b_ref, gam_ref, bet_ref,
                 x_ref, o_ref):
    tot = g_ref[...] + acc_ref[0, :N, :] + acc_ref[1, :N, :]
    tot = tot * dinv_ref[...] + b_ref[...]
    mu = jnp.mean(tot, axis=1, keepdims=True)
    cen = tot - mu
    var = jnp.mean(cen * cen, axis=1, keepdims=True)
    xh = cen * lax.rsqrt(var + EPS)
    y = xh * gam_ref[...] + bet_ref[...]
    o_ref[...] = jnp.maximum(y + x_ref[...], 0.0)


def _tc_epilogue(g, acc, dinv, b, gamma, beta, x):
    return pl.pallas_call(
        _tc_epi_body,
        out_shape=jax.ShapeDtypeStruct((N, D), jnp.float32),
    )(g, acc, dinv, b.reshape(1, D), gamma.reshape(1, D),
      beta.reshape(1, D), x)


def kernel(x, edge_index, W, b, gamma, beta):
    pad = EP - E
    ar = jnp.arange(pad, dtype=jnp.int32)
    src_pad = jnp.concatenate([edge_index[0], (ar * 131) % N])
    dst_pad = jnp.concatenate([edge_index[1], N + ar % (NP - N)])
    src2 = src_pad.reshape(NW * CH, CW)
    dst2 = dst_pad.reshape(NW * CH, CW)
    counts = _sc_count(dst2).reshape(NW, NP)
    h = _tc_mm(x, W)
    g, dinv = _tc_scale(h, counts)
    acc = _sc_scatter(src2, dst2, g)
    return _tc_epilogue(g, acc, dinv, b, gamma, beta, x)
